# baseline (speedup 1.0000x reference)
import functools

import jax
import jax.numpy as jnp
from jax import lax
from jax.experimental import pallas as pl
from jax.experimental.pallas import tpu as pltpu
from jax.experimental.pallas import tpu_sc as plsc

B = 16384
NW = 16
ROWS_W = B // NW
QTR = 128
L = 16
D_OUT = 148

_DENSE = ((12, 0, 43691, 19), (3, 12, 43691, 17), (13, 15, 20165, 18),
          (12, 72, 43691, 19), (3, 84, 43691, 17), (13, 87, 20165, 18))
_EMB = ((0, 32, 28), (1, 4, 60), (2, 8, 64),
        (0, 32, 100), (1, 4, 132), (2, 8, 136), (3, 4, 144))

_SET = [pltpu.VMEM((QTR * w,), jnp.float32) for w, _, _, _ in _DENSE] + [
    pltpu.VMEM((QTR, D_OUT), jnp.float32),
    pltpu.SemaphoreType.DMA,
    pltpu.SemaphoreType.DMA,
]
_NS = len(_SET)


@functools.partial(
    pl.kernel,
    out_type=jax.ShapeDtypeStruct((B, D_OUT), jnp.float32),
    mesh=plsc.VectorSubcoreMesh(core_axis_name="c", subcore_axis_name="s",
                                num_cores=1),
    compiler_params=pltpu.CompilerParams(
        use_tc_tiling_on_sc=True, needs_layout_passes=False),
    scratch_types=[
        pltpu.VMEM((7 * ROWS_W,), jnp.int32),
        pltpu.VMEM((400 * 32,), jnp.float32),
        pltpu.VMEM((8 * 4,), jnp.float32),
        pltpu.VMEM((33 * 8,), jnp.float32),
        pltpu.VMEM((33 * 4,), jnp.float32),
        pltpu.SemaphoreType.DMA,
    ] + _SET + _SET,
)
def _encode_sc(p0c, p0b, p0k, p1c, p1b, p1k,
               i_p0a, i_p0j, i_p0c, i_p1a, i_p1j, i_p1c, i_stg,
               t_act, t_jmp, t_chr, t_stg,
               out_hbm,
               idxv, va, vj, vc, vs, psem, *sets):
    setA, setB = sets[:_NS], sets[_NS:]
    vtabs = (va, vj, vc, vs)
    twidth = (32, 4, 8, 4)
    wid = lax.axis_index("s")
    base = wid * ROWS_W

    pcps = []
    for src, dst in zip((t_act, t_jmp, t_chr, t_stg), vtabs):
        pcps.append(pltpu.async_copy(src, dst, psem))
    for f, ih in enumerate((i_p0a, i_p0j, i_p0c, i_p1a, i_p1j, i_p1c, i_stg)):
        pcps.append(pltpu.async_copy(
            ih.at[pl.ds(base, ROWS_W)],
            idxv.at[pl.ds(f * ROWS_W, ROWS_W)], psem))
    for cp in pcps:
        cp.wait()

    dsrcs = (p0c, p0b, p0k, p1c, p1b, p1k)
    iota = lax.iota(jnp.int32, L)

    def fire(q, S):
        gsem = S[_NS - 2]
        return [pltpu.async_copy(
                    dsrc.at[pl.ds((base + q * QTR) * w, QTR * w)],
                    S[i], gsem)
                for i, (dsrc, (w, _, _, _)) in enumerate(zip(dsrcs, _DENSE))]

    def assemble(q, S):
        outb = S[6]

        for i, (w, off, mg, sh) in enumerate(_DENSE):
            sref = S[i]

            @plsc.parallel_loop(0, QTR * w // L, unroll=8)
            def dense_chunk(k, sref=sref, w=w, off=off, mg=mg, sh=sh):
                svec = k * L + iota
                rvec = lax.shift_right_logical(svec * mg, sh)
                cvec = svec - rvec * w
                vals = plsc.load_gather(sref, [svec])
                plsc.store_scatter(outb, [rvec, cvec + off], vals)

        for f, (tid, w, off) in enumerate(_EMB):
            tab = vtabs[tid]
            tw = twidth[tid]
            lw = w.bit_length() - 1
            fbase = f * ROWS_W + q * QTR

            @plsc.parallel_loop(0, QTR * w // L, unroll=8)
            def emb_chunk(k, tab=tab, tw=tw, w=w, off=off, lw=lw, fbase=fbase):
                svec = k * L + iota
                rvec = lax.shift_right_logical(svec, lw)
                cvec = lax.bitwise_and(svec, w - 1)
                ivec = plsc.load_gather(idxv, [rvec + fbase])
                vals = plsc.load_gather(tab, [ivec * tw + cvec])
                plsc.store_scatter(outb, [rvec, cvec + off], vals)

    nq = ROWS_W // QTR
    stage_cps = {0: fire(0, setA)}
    wb = {}
    for q in range(nq):
        S = (setA, setB)[q % 2]
        if q + 1 < nq:
            stage_cps[q + 1] = fire(q + 1, (setA, setB)[(q + 1) % 2])
        for cp in stage_cps.pop(q):
            cp.wait()
        if q >= 2:
            wb[q - 2].wait()
        assemble(q, S)
        wb[q] = pltpu.async_copy(
            S[6], out_hbm.at[pl.ds(base + q * QTR, QTR), :], S[_NS - 1])
    wb[nq - 2].wait()
    wb[nq - 1].wait()


def kernel(p0_continuous, p0_binary, p0_controller, p0_action, p0_jumps,
           p0_character, p1_continuous, p1_binary, p1_controller, p1_action,
           p1_jumps, p1_character, stage, action_table, jumps_table,
           char_table, stage_table):
    def idx(a):
        return a.astype(jnp.int32)
    return _encode_sc(
        p0_continuous.reshape(-1), p0_binary.reshape(-1),
        p0_controller.reshape(-1), p1_continuous.reshape(-1),
        p1_binary.reshape(-1), p1_controller.reshape(-1),
        idx(p0_action), idx(p0_jumps), idx(p0_character),
        idx(p1_action), idx(p1_jumps), idx(p1_character), idx(stage),
        action_table.reshape(-1), jumps_table.reshape(-1),
        char_table.reshape(-1), stage_table.reshape(-1))


# speedup vs baseline: 1.8467x; 1.8467x over previous
import functools

import jax
import jax.numpy as jnp
from jax import lax
from jax.experimental import pallas as pl
from jax.experimental.pallas import tpu as pltpu
from jax.experimental.pallas import tpu_sc as plsc

B = 16384
NW = 32
ROWS_W = B // NW
QTR = 128
L = 16
D_OUT = 148

_DENSE = ((12, 0, 43691, 19), (3, 12, 43691, 17), (13, 15, 20165, 18),
          (12, 72, 43691, 19), (3, 84, 43691, 17), (13, 87, 20165, 18))
_EMB = ((0, 32, 28), (1, 4, 60), (2, 8, 64),
        (0, 32, 100), (1, 4, 132), (2, 8, 136), (3, 4, 144))

_SET = [pltpu.VMEM((w, QTR), jnp.float32) for w, _, _, _ in _DENSE] + [
    pltpu.VMEM((QTR, D_OUT), jnp.float32),
    pltpu.SemaphoreType.DMA,
    pltpu.SemaphoreType.DMA,
]
_NS = len(_SET)


@functools.partial(
    pl.kernel,
    out_type=jax.ShapeDtypeStruct((B, D_OUT), jnp.float32),
    mesh=plsc.VectorSubcoreMesh(core_axis_name="c", subcore_axis_name="s"),
    compiler_params=pltpu.CompilerParams(
        use_tc_tiling_on_sc=True, needs_layout_passes=False),
    scratch_types=[
        pltpu.VMEM((7 * ROWS_W,), jnp.int32),
        pltpu.VMEM((32, 400), jnp.float32),
        pltpu.VMEM((4, 8), jnp.float32),
        pltpu.VMEM((8, 33), jnp.float32),
        pltpu.VMEM((4, 33), jnp.float32),
        pltpu.SemaphoreType.DMA,
    ] + _SET + _SET,
)
def _encode_sc(p0c, p0b, p0k, p1c, p1b, p1k,
               i_p0a, i_p0j, i_p0c, i_p1a, i_p1j, i_p1c, i_stg,
               t_act, t_jmp, t_chr, t_stg,
               out_hbm,
               idxv, va, vj, vc, vs, psem, *sets):
    setA, setB = sets[:_NS], sets[_NS:]
    vtabs = (va, vj, vc, vs)
    wid = lax.axis_index("s") * 2 + lax.axis_index("c")
    base = wid * ROWS_W

    pcps = []
    for src, dst in zip((t_act, t_jmp, t_chr, t_stg), vtabs):
        pcps.append(pltpu.async_copy(src, dst, psem))
    for f, ih in enumerate((i_p0a, i_p0j, i_p0c, i_p1a, i_p1j, i_p1c, i_stg)):
        pcps.append(pltpu.async_copy(
            ih.at[pl.ds(base, ROWS_W)],
            idxv.at[pl.ds(f * ROWS_W, ROWS_W)], psem))
    for cp in pcps:
        cp.wait()

    dsrcs = (p0c, p0b, p0k, p1c, p1b, p1k)
    iota = lax.iota(jnp.int32, L)

    def fire(q, S):
        gsem = S[_NS - 2]
        return [pltpu.async_copy(
                    dsrc.at[:, pl.ds(base + q * QTR, QTR)], S[i], gsem)
                for i, dsrc in enumerate(dsrcs)]

    def assemble(q, S):
        outb = S[6]

        for i, (w, off, mg, sh) in enumerate(_DENSE):
            sref = S[i]

            @plsc.parallel_loop(0, QTR * w // L, unroll=8)
            def dense_chunk(k, sref=sref, w=w, off=off, mg=mg, sh=sh):
                svec = k * L + iota
                rvec = lax.shift_right_logical(svec * mg, sh)
                cvec = svec - rvec * w
                vals = plsc.load_gather(sref, [cvec, rvec])
                plsc.store_scatter(outb, [rvec, cvec + off], vals)

        for f, (tid, w, off) in enumerate(_EMB):
            tab = vtabs[tid]
            lw = w.bit_length() - 1
            fbase = f * ROWS_W + q * QTR

            @plsc.parallel_loop(0, QTR * w // L, unroll=8)
            def emb_chunk(k, tab=tab, w=w, off=off, lw=lw, fbase=fbase):
                svec = k * L + iota
                rvec = lax.shift_right_logical(svec, lw)
                cvec = lax.bitwise_and(svec, w - 1)
                ivec = plsc.load_gather(idxv, [rvec + fbase])
                vals = plsc.load_gather(tab, [cvec, ivec])
                plsc.store_scatter(outb, [rvec, cvec + off], vals)

    nq = ROWS_W // QTR
    stage_cps = {0: fire(0, setA)}
    wb = {}
    for q in range(nq):
        S = (setA, setB)[q % 2]
        if q + 1 < nq:
            stage_cps[q + 1] = fire(q + 1, (setA, setB)[(q + 1) % 2])
        for cp in stage_cps.pop(q):
            cp.wait()
        if q >= 2:
            wb[q - 2].wait()
        assemble(q, S)
        wb[q] = pltpu.async_copy(
            S[6], out_hbm.at[pl.ds(base + q * QTR, QTR), :], S[_NS - 1])
    wb[nq - 2].wait()
    wb[nq - 1].wait()


def kernel(p0_continuous, p0_binary, p0_controller, p0_action, p0_jumps,
           p0_character, p1_continuous, p1_binary, p1_controller, p1_action,
           p1_jumps, p1_character, stage, action_table, jumps_table,
           char_table, stage_table):
    def idx(a):
        return a.astype(jnp.int32)
    return _encode_sc(
        p0_continuous.T, p0_binary.T, p0_controller.T,
        p1_continuous.T, p1_binary.T, p1_controller.T,
        idx(p0_action), idx(p0_jumps), idx(p0_character),
        idx(p1_action), idx(p1_jumps), idx(p1_character), idx(stage),
        action_table.T, jumps_table.T, char_table.T, stage_table.T)
